# Initial kernel scaffold; baseline (speedup 1.0000x reference)
import jax
import jax.numpy as jnp
from jax.experimental import pallas as pl


def kernel(x, temperature, proj_W, proj_b, sim_matrix, A_w, A_b, B_w, B_b):
    raise NotImplementedError("write your pallas kernel here")


# speedup vs baseline: 1.0888x; 1.0888x over previous
import functools

import jax
import jax.numpy as jnp
import numpy as np
from jax import lax
from jax.experimental import pallas as pl
from jax.experimental.pallas import tpu as pltpu

D_MODEL = 1024
N_EXPERTS = 8
TOP_K = 2
EXPERT_DIM = 2048
PROJ_DIM = 256
T_TOKENS = 2048
M_TILE = 256


def _router_kernel(x_ref, pw_ref, pb_ref, sim_ref, scale_ref,
                   combine_ref, csum_ref, colsum_ref, usage_ref):
    x = x_ref[...]
    proj = lax.dot_general(x, pw_ref[...], (((1,), (1,)), ((), ())),
                           preferred_element_type=jnp.float32)
    proj = proj + pb_ref[...]
    norm = jnp.sqrt(jnp.sum(proj * proj, axis=1, keepdims=True))
    proj_n = proj / jnp.maximum(norm, 1e-12)
    sim = sim_ref[...]
    sim_norm = jnp.sqrt(jnp.sum(sim * sim, axis=0, keepdims=True))
    sim_n = sim / jnp.maximum(sim_norm, 1e-12)
    logits = lax.dot_general(proj_n, sim_n, (((1,), (0,)), ((), ())),
                             preferred_element_type=jnp.float32)
    gate = logits * scale_ref[0, 0]
    m = jnp.max(gate, axis=1, keepdims=True)
    p = jnp.exp(gate - m)
    probs = p / jnp.sum(p, axis=1, keepdims=True)

    iota = lax.broadcasted_iota(jnp.int32, (T_TOKENS, N_EXPERTS), 1)
    m1 = jnp.max(probs, axis=1, keepdims=True)
    i1 = jnp.min(jnp.where(probs == m1, iota, N_EXPERTS), axis=1, keepdims=True)
    oh1 = (iota == i1).astype(jnp.float32)
    rest = jnp.where(iota == i1, -1.0, probs)
    m2 = jnp.max(rest, axis=1, keepdims=True)
    i2 = jnp.min(jnp.where(rest == m2, iota, N_EXPERTS), axis=1, keepdims=True)
    oh2 = (iota == i2).astype(jnp.float32)

    s = m1 + m2 + 1e-8
    pr1 = m1 / s
    pr2 = m2 / s
    combine_ref[...] = oh1 * pr1 + oh2 * pr2
    csum_ref[...] = pr1 + pr2
    colsum_ref[...] = jnp.sum(probs, axis=0, keepdims=True)
    usage_ref[...] = jnp.sum(oh1 + oh2, axis=0, keepdims=True)


def _router(xf, proj_W, proj_b, sim_matrix, scale):
    return pl.pallas_call(
        _router_kernel,
        out_shape=(
            jax.ShapeDtypeStruct((T_TOKENS, N_EXPERTS), jnp.float32),
            jax.ShapeDtypeStruct((T_TOKENS, 1), jnp.float32),
            jax.ShapeDtypeStruct((1, N_EXPERTS), jnp.float32),
            jax.ShapeDtypeStruct((1, N_EXPERTS), jnp.float32),
        ),
    )(xf, proj_W, proj_b.reshape(1, PROJ_DIM), sim_matrix,
      scale.reshape(1, 1))


def _expert_kernel(x_ref, combine_ref, csum_ref,
                   aw_ref, ab_ref, bw_ref, bb_ref, out_ref):
    e = pl.program_id(0)
    m = pl.program_id(1)
    rows = pl.ds(m * M_TILE, M_TILE)
    xt = x_ref[rows, :]
    hidden = lax.dot_general(xt, aw_ref[0], (((1,), (1,)), ((), ())),
                             preferred_element_type=jnp.float32)
    hidden = hidden + ab_ref[0]
    adapter = lax.dot_general(hidden, bw_ref[0], (((1,), (1,)), ((), ())),
                              preferred_element_type=jnp.float32)
    adapter = adapter + bb_ref[0]
    lane = lax.broadcasted_iota(jnp.int32, (M_TILE, N_EXPERTS), 1)
    c_e = jnp.sum(combine_ref[rows, :] * (lane == e).astype(jnp.float32),
                  axis=1, keepdims=True)
    contrib = c_e * adapter

    @pl.when(e == 0)
    def _():
        out_ref[rows, :] = csum_ref[rows, :] * xt + contrib

    @pl.when(e > 0)
    def _():
        out_ref[rows, :] = out_ref[rows, :] + contrib


def _experts(xf, combine, csum, A_w, A_b, B_w, B_b):
    n_m = T_TOKENS // M_TILE
    return pl.pallas_call(
        _expert_kernel,
        grid=(N_EXPERTS, n_m),
        in_specs=[
            pl.BlockSpec((T_TOKENS, D_MODEL), lambda e, m: (0, 0)),
            pl.BlockSpec((T_TOKENS, N_EXPERTS), lambda e, m: (0, 0)),
            pl.BlockSpec((T_TOKENS, 1), lambda e, m: (0, 0)),
            pl.BlockSpec((1, EXPERT_DIM, D_MODEL), lambda e, m: (e, 0, 0)),
            pl.BlockSpec((1, 1, EXPERT_DIM), lambda e, m: (e, 0, 0)),
            pl.BlockSpec((1, D_MODEL, EXPERT_DIM), lambda e, m: (e, 0, 0)),
            pl.BlockSpec((1, 1, D_MODEL), lambda e, m: (e, 0, 0)),
        ],
        out_specs=pl.BlockSpec((T_TOKENS, D_MODEL), lambda e, m: (0, 0)),
        out_shape=jax.ShapeDtypeStruct((T_TOKENS, D_MODEL), jnp.float32),
    )(xf, combine, csum, A_w, A_b.reshape(N_EXPERTS, 1, EXPERT_DIM),
      B_w, B_b.reshape(N_EXPERTS, 1, D_MODEL))


def kernel(x, temperature, proj_W, proj_b, sim_matrix, A_w, A_b, B_w, B_b):
    Bsz, S, D = x.shape
    xf = x.reshape(-1, D)
    clamp_max = np.log(1.0 / 0.01)
    scale = jnp.exp(jnp.minimum(temperature, clamp_max))

    combine, csum, colsum, usage = _router(xf, proj_W, proj_b, sim_matrix,
                                           scale)
    out_flat = _experts(xf, combine, csum, A_w, A_b, B_w, B_b)

    t = float(T_TOKENS)
    frac = colsum[0] / t
    aux_loss = jnp.sum((frac - 1.0 / N_EXPERTS) ** 2)
    gate_probs_mean = colsum[0] / t
    expert_usage = usage[0]
    return (out_flat.reshape(Bsz, S, D), aux_loss, gate_probs_mean,
            expert_usage)
